# baseline (speedup 1.0000x reference)
import functools

import jax
import jax.numpy as jnp
from jax import lax
from jax.experimental import pallas as pl
from jax.experimental.pallas import tpu as pltpu
from jax.experimental.pallas import tpu_sc as plsc

BATCH = 1024
HIST = 200
D = 128
B = BATCH * HIST
NC = 2
NS = 16
NW = NC * NS
BPW = B // NW
CH = 128
NCH = BPW // CH
PL = 2
NBIG = NCH // PL
NBUF = 3
NGRP = NBIG // NBUF

_mesh = plsc.VectorSubcoreMesh(core_axis_name="c", subcore_axis_name="s")


def _gather_chunk(tab, idx_v, bufs, b, c, gsem):
    for p in range(PL):
        pltpu.async_copy(tab.at[idx_v.at[c * PL + p]], bufs.at[b].at[p], gsem)


def _wait_chunk_gathers(tab, idx_v, bufs, b, gsem):
    for p in range(PL):
        pltpu.make_async_copy(
            tab.at[idx_v.at[p]], bufs.at[b].at[p], gsem).wait()


@functools.partial(
    pl.kernel,
    out_type=[
        jax.ShapeDtypeStruct((B // CH, CH, D), jnp.float32),
        jax.ShapeDtypeStruct((B // CH, CH, D), jnp.float32),
    ],
    mesh=_mesh,
    scratch_types=[
        pltpu.VMEM((NCH, CH), jnp.int32),
        pltpu.VMEM((NBUF, PL, CH, D), jnp.float32),
        pltpu.SemaphoreType.DMA,
        pltpu.SemaphoreType.DMA,
    ],
)
def _emb_gather(eid_hbm, wid_hbm, etab_hbm, wtab_hbm, eout_hbm, wout_hbm,
                idx_v, bufs, gsem, ssem):
    wid = lax.axis_index("s") * NC + lax.axis_index("c")
    pbase = wid * NCH

    for ids3d, tab, out in ((eid_hbm, etab_hbm, eout_hbm),
                            (wid_hbm, wtab_hbm, wout_hbm)):
        pltpu.sync_copy(ids3d.at[wid], idx_v)

        for b in range(NBUF):
            _gather_chunk(tab, idx_v, bufs, b, b, gsem)

        @pl.loop(0, NGRP)
        def _grp(g, tab=tab, out=out):
            for b in range(NBUF):
                c = g * NBUF + b
                _wait_chunk_gathers(tab, idx_v, bufs, b, gsem)
                pltpu.async_copy(
                    bufs.at[b], out.at[pl.ds(pbase + c * PL, PL)], ssem)
            for b in range(NBUF):
                pltpu.make_async_copy(
                    bufs.at[b], out.at[pl.ds(pbase, PL)], ssem).wait()

                @pl.when((g + 1) * NBUF + b < NBIG)
                def _(b=b):
                    _gather_chunk(tab, idx_v, bufs, b, (g + 1) * NBUF + b,
                                  gsem)

        for t in range(NGRP * NBUF, NBIG):
            b = t % NBUF
            _wait_chunk_gathers(tab, idx_v, bufs, b, gsem)
            pltpu.async_copy(
                bufs.at[b], out.at[pl.ds(pbase + t * PL, PL)], ssem)
            pltpu.make_async_copy(
                bufs.at[b], out.at[pl.ds(pbase, PL)], ssem).wait()


def kernel(event_ids, word_ids, event_table, word_table):
    eid = event_ids.astype(jnp.int32).reshape(NW, NCH, CH)
    wid = word_ids.astype(jnp.int32).reshape(NW, NCH, CH)
    eout, wout = _emb_gather(eid, wid, event_table, word_table)
    return (eout.reshape(BATCH, HIST, D), wout.reshape(BATCH, HIST, D))


# speedup vs baseline: 1.0058x; 1.0058x over previous
import functools

import jax
import jax.numpy as jnp
from jax import lax
from jax.experimental import pallas as pl
from jax.experimental.pallas import tpu as pltpu
from jax.experimental.pallas import tpu_sc as plsc

BATCH = 1024
HIST = 200
D = 128
B = BATCH * HIST
NC = 2
NS = 16
NW = NC * NS
BPW = B // NW
CH = 128
NCH = BPW // CH
NBUF = 5
NGRP = NCH // NBUF

_mesh = plsc.VectorSubcoreMesh(core_axis_name="c", subcore_axis_name="s")


@functools.partial(
    pl.kernel,
    out_type=[
        jax.ShapeDtypeStruct((B, D), jnp.float32),
        jax.ShapeDtypeStruct((B, D), jnp.float32),
    ],
    mesh=_mesh,
    scratch_types=[
        pltpu.VMEM((2, NCH, CH), jnp.int32),
        pltpu.VMEM((NBUF, CH, D), jnp.float32),
        pltpu.SemaphoreType.DMA,
        pltpu.SemaphoreType.DMA,
    ],
)
def _emb_gather(eid_hbm, wid_hbm, etab_hbm, wtab_hbm, eout_hbm, wout_hbm,
                idx_v, bufs, gsem, ssem):
    wid = lax.axis_index("s") * NC + lax.axis_index("c")
    base = wid * BPW

    pltpu.async_copy(eid_hbm.at[wid], idx_v.at[0], gsem)
    pltpu.async_copy(wid_hbm.at[wid], idx_v.at[1], gsem)
    pltpu.make_async_copy(eid_hbm.at[wid], idx_v.at[0], gsem).wait()
    pltpu.make_async_copy(wid_hbm.at[wid], idx_v.at[1], gsem).wait()

    for b in range(NBUF):
        pltpu.async_copy(etab_hbm.at[idx_v.at[0].at[b]], bufs.at[b], gsem)

    for t, (tab, out, ntab) in enumerate(((etab_hbm, eout_hbm, wtab_hbm),
                                          (wtab_hbm, wout_hbm, None))):
        @pl.loop(0, NGRP)
        def _grp(g, t=t, tab=tab, out=out, ntab=ntab):
            for b in range(NBUF):
                c = g * NBUF + b
                pltpu.make_async_copy(
                    tab.at[idx_v.at[t].at[b]], bufs.at[b], gsem).wait()
                pltpu.async_copy(
                    bufs.at[b], out.at[pl.ds(base + c * CH, CH)], ssem)
            for b in range(NBUF):
                pltpu.make_async_copy(
                    bufs.at[b], out.at[pl.ds(base, CH)], ssem).wait()

                @pl.when(g + 1 < NGRP)
                def _(b=b):
                    c = (g + 1) * NBUF + b
                    pltpu.async_copy(tab.at[idx_v.at[t].at[c]],
                                     bufs.at[b], gsem)

                if ntab is not None:
                    @pl.when(g + 1 == NGRP)
                    def _(b=b):
                        pltpu.async_copy(ntab.at[idx_v.at[t + 1].at[b]],
                                         bufs.at[b], gsem)


def kernel(event_ids, word_ids, event_table, word_table):
    eid = event_ids.astype(jnp.int32).reshape(NW, NCH, CH)
    wid = word_ids.astype(jnp.int32).reshape(NW, NCH, CH)
    eout, wout = _emb_gather(eid, wid, event_table, word_table)
    return (eout.reshape(BATCH, HIST, D), wout.reshape(BATCH, HIST, D))
